# Initial kernel scaffold; baseline (speedup 1.0000x reference)
import jax
import jax.numpy as jnp
from jax.experimental import pallas as pl


def kernel(logits, hidden_states, draft_tokens, slot_ids, pool_hidden, pool_tokens, embed_tokens, lm_head, W_mtp):
    raise NotImplementedError("write your pallas kernel here")


# speedup vs baseline: 5.4807x; 5.4807x over previous
import functools

import jax
import jax.numpy as jnp
from jax import lax
from jax.experimental import pallas as pl
from jax.experimental.pallas import tpu as pltpu
from jax.experimental.pallas import tpu_sc as plsc

B = 64
K = 3
D = 2048
V = 32000
M = 256
R = B * (K + 1)
PK = M * K
CAT = PK + R

NC = 2
NS = 16
NW = NC * NS


_VT_A = 6400


def _argmax_logits_body(lg_ref, out_ref, mx_ref, am_ref):
    i = pl.program_id(0)

    @pl.when(i == 0)
    def _init():
        mx_ref[...] = jnp.full((R, 1), -jnp.inf, dtype=jnp.float32)
        am_ref[...] = jnp.zeros((R, 1), dtype=jnp.int32)

    blk = lg_ref[...]
    tm = jnp.max(blk, axis=1, keepdims=True)
    ta = jnp.argmax(blk, axis=1).astype(jnp.int32)[:, None] + i * _VT_A
    upd = tm > mx_ref[...]
    am_ref[...] = jnp.where(upd, ta, am_ref[...])
    mx_ref[...] = jnp.where(upd, tm, mx_ref[...])
    out_ref[...] = am_ref[...]


def _argmax_logits(logits):
    return pl.pallas_call(
        _argmax_logits_body,
        grid=(V // _VT_A,),
        in_specs=[pl.BlockSpec((R, _VT_A), lambda i: (0, i))],
        out_specs=pl.BlockSpec((R, 1), lambda i: (0, 0)),
        out_shape=jax.ShapeDtypeStruct((R, 1), jnp.int32),
        scratch_shapes=[
            pltpu.VMEM((R, 1), jnp.float32),
            pltpu.VMEM((R, 1), jnp.int32),
        ],
    )(logits)


_mesh = plsc.VectorSubcoreMesh(core_axis_name="c", subcore_axis_name="s")
_sc_params = pltpu.CompilerParams(needs_layout_passes=False)


def _iota16():
    return lax.iota(jnp.int32, 16)


@functools.partial(
    pl.kernel,
    mesh=_mesh,
    compiler_params=_sc_params,
    out_type=[
        jax.ShapeDtypeStruct((B,), jnp.int32),
        jax.ShapeDtypeStruct((B,), jnp.int32),
        jax.ShapeDtypeStruct((PK,), jnp.int32),
        jax.ShapeDtypeStruct((PK,), jnp.int32),
        jax.ShapeDtypeStruct((B, D), jnp.float32),
    ],
    scratch_types=[
        pltpu.VMEM((CAT,), jnp.int32),
        pltpu.VMEM((K, B), jnp.int32),
        pltpu.VMEM((B,), jnp.int32),
        pltpu.VMEM((B,), jnp.int32),
        pltpu.VMEM((B,), jnp.int32),
        pltpu.VMEM((B,), jnp.int32),
        pltpu.VMEM((PK,), jnp.int32),
        pltpu.VMEM((PK,), jnp.int32),
        pltpu.VMEM((16, D), jnp.float32),
        pltpu.SemaphoreType.DMA,
    ],
)
def _sc_accept(cat_t_hbm, draft_t_hbm, slot_hbm, cat_h_hbm,
               na_out, ctk_out, npt_out, src_out, curh_out,
               ct_v, dr_v, sl_v, na_v, ctk_v, hr_v, src_v, npt_v,
               rows_v, sem):
    wid = lax.axis_index("c") * NS + lax.axis_index("s")

    @pl.when(wid < 4)
    def _active():
        pltpu.sync_copy(cat_t_hbm, ct_v)
        pltpu.sync_copy(draft_t_hbm, dr_v)
        pltpu.sync_copy(slot_hbm, sl_v)

        for c in range(B // 16):
            bidx = _iota16() + 16 * c
            row0 = PK + 4 * bidx
            t = [plsc.load_gather(ct_v, [row0 + k]) for k in range(K + 1)]
            d = [dr_v[k, pl.ds(16 * c, 16)] for k in range(K)]
            m = jnp.int32(1)
            na = jnp.full((16,), 1, dtype=jnp.int32)
            for k in range(K):
                m = m * jnp.where(t[k] == d[k], 1, 0)
                na = na + m
            na_v[pl.ds(16 * c, 16)] = na
            hrow = row0 + na - 1
            hr_v[pl.ds(16 * c, 16)] = hrow
            ctk_v[pl.ds(16 * c, 16)] = plsc.load_gather(ct_v, [hrow])

        @pl.when(wid == 0)
        def _tile0():
            for c in range(PK // 16):
                src_v[pl.ds(16 * c, 16)] = _iota16() + 16 * c
            for c in range(B // 16):
                bidx = _iota16() + 16 * c
                sl = sl_v[pl.ds(16 * c, 16)]
                na = na_v[pl.ds(16 * c, 16)]
                for j in range(K):
                    srcp = na + j
                    val = jnp.where(srcp < K, K * sl + srcp,
                                    PK + 4 * bidx + srcp - K)
                    plsc.store_scatter(src_v, [K * sl + j], val)
            for c in range(PK // 16):
                idx = src_v[pl.ds(16 * c, 16)]
                npt_v[pl.ds(16 * c, 16)] = plsc.load_gather(ct_v, [idx])
            pltpu.sync_copy(na_v, na_out)
            pltpu.sync_copy(ctk_v, ctk_out)
            pltpu.sync_copy(npt_v, npt_out)
            pltpu.sync_copy(src_v, src_out)

        base = wid * 16
        idxvec = hr_v[pl.ds(base, 16)]
        pltpu.async_copy(cat_h_hbm.at[idxvec], rows_v, sem).wait()
        pltpu.sync_copy(rows_v, curh_out.at[pl.ds(base, 16)])


_RPT = PK // NW


@functools.partial(
    pl.kernel,
    mesh=_mesh,
    compiler_params=_sc_params,
    out_type=jax.ShapeDtypeStruct((PK, D), jnp.float32),
    scratch_types=[
        pltpu.VMEM((_RPT,), jnp.int32),
        pltpu.VMEM((_RPT, D), jnp.float32),
        pltpu.SemaphoreType.DMA,
    ],
)
def _sc_pool_hidden(cat_h_hbm, src_hbm, out_hbm, idx_v, buf_v, sem):
    wid = lax.axis_index("c") * NS + lax.axis_index("s")
    base = wid * _RPT
    pltpu.sync_copy(src_hbm.at[pl.ds(base, _RPT)], idx_v)
    pltpu.async_copy(cat_h_hbm.at[idx_v], buf_v, sem).wait()
    pltpu.sync_copy(buf_v, out_hbm.at[pl.ds(base, _RPT)])


_VT = 1280
_NL = V // _VT


def _mega_body(tt_sref, dr_sref, w_ref, lm_ref, emb_any, hs_any,
               tokout_ref,
               emb_scr, h_scr, mx_ref, am_ref, tok_smem, sem, csem):
    k = pl.program_id(0)
    t = pl.program_id(1)

    @pl.when(t == 0)
    def _proj_phase():
        @pl.when(k == 0)
        def _first():
            for b in range(B):
                m = jnp.int32(1)
                na = jnp.int32(1)
                for kk in range(K):
                    m = m * jnp.where(
                        tt_sref[(K + 1) * b + kk] == dr_sref[K * b + kk], 1, 0)
                    na = na + m
                rowh = (K + 1) * b + na - 1
                pltpu.make_async_copy(
                    hs_any.at[pl.ds(rowh, 1)], h_scr.at[pl.ds(b, 1)], csem
                ).start()
                pltpu.make_async_copy(
                    emb_any.at[pl.ds(tt_sref[rowh], 1)],
                    emb_scr.at[pl.ds(b, 1)], sem
                ).start()
            for b in range(B):
                pltpu.make_async_copy(
                    hs_any.at[pl.ds(0, 1)], h_scr.at[pl.ds(b, 1)], csem
                ).wait()
                pltpu.make_async_copy(
                    emb_any.at[pl.ds(0, 1)], emb_scr.at[pl.ds(b, 1)], sem
                ).wait()

        @pl.when(k > 0)
        def _handoff():
            pltpu.make_async_copy(am_ref, tok_smem, csem).start()
            pltpu.make_async_copy(am_ref, tok_smem, csem).wait()
            for b in range(B):
                pltpu.make_async_copy(
                    emb_any.at[pl.ds(tok_smem[b, 0], 1)],
                    emb_scr.at[pl.ds(b, 1)], sem
                ).start()
            for b in range(B):
                pltpu.make_async_copy(
                    emb_any.at[pl.ds(0, 1)], emb_scr.at[pl.ds(b, 1)], sem
                ).wait()

        x = (jnp.dot(emb_scr[...], w_ref[:D, :],
                     preferred_element_type=jnp.float32)
             + jnp.dot(h_scr[...], w_ref[D:, :],
                       preferred_element_type=jnp.float32))
        ms = jnp.mean(x * x, axis=1, keepdims=True)
        hn = x * lax.rsqrt(ms + 1e-6)
        h_scr[...] = hn
        mx_ref[...] = jnp.full((B, 1), -jnp.inf, dtype=jnp.float32)
        am_ref[...] = jnp.zeros((B, 1), dtype=jnp.int32)

    @pl.when(t > 0)
    def _lm_phase():
        i = t - 1
        dl = jnp.dot(h_scr[...], lm_ref[...],
                     preferred_element_type=jnp.float32)
        tm = jnp.max(dl, axis=1, keepdims=True)
        ta = jnp.argmax(dl, axis=1).astype(jnp.int32)[:, None] + i * _VT
        upd = tm > mx_ref[...]
        am_ref[...] = jnp.where(upd, ta, am_ref[...])
        mx_ref[...] = jnp.where(upd, tm, mx_ref[...])
        tokout_ref[0, :, :] = am_ref[...]

    @pl.when(t == _NL)
    def _write_tok():
        tokout_ref[0, :, :] = am_ref[...]


def _drafter_all(target, draft, w_mtp, lm_head, embed_tokens, hidden_states):
    grid_spec = pltpu.PrefetchScalarGridSpec(
        num_scalar_prefetch=2,
        grid=(K, 1 + _NL),
        in_specs=[
            pl.BlockSpec((2 * D, D), lambda k, t, tt, dr: (0, 0)),
            pl.BlockSpec((D, _VT),
                         lambda k, t, tt, dr: (0, jnp.maximum(t - 1, 0))),
            pl.BlockSpec(memory_space=pl.ANY),
            pl.BlockSpec(memory_space=pl.ANY),
        ],
        out_specs=[
            pl.BlockSpec((1, B, 1), lambda k, t, tt, dr: (k, 0, 0)),
        ],
        scratch_shapes=[
            pltpu.VMEM((B, D), jnp.float32),
            pltpu.VMEM((B, D), jnp.float32),
            pltpu.VMEM((B, 1), jnp.float32),
            pltpu.VMEM((B, 1), jnp.int32),
            pltpu.SMEM((B, 1), jnp.int32),
            pltpu.SemaphoreType.DMA,
            pltpu.SemaphoreType.DMA,
        ],
    )
    toks = pl.pallas_call(
        _mega_body,
        grid_spec=grid_spec,
        out_shape=[
            jax.ShapeDtypeStruct((K, B, 1), jnp.int32),
        ],
    )(target, draft, w_mtp, lm_head, embed_tokens, hidden_states)
    return toks[0]


def kernel(logits, hidden_states, draft_tokens, slot_ids, pool_hidden,
           pool_tokens, embed_tokens, lm_head, W_mtp):
    target = _argmax_logits(logits)
    accepted_tokens = target.reshape(B, K + 1)

    cat_t = jnp.concatenate([pool_tokens.reshape(PK), target.reshape(R)])
    cat_h = jnp.concatenate([pool_hidden.reshape(PK, D), hidden_states])
    draft_t = draft_tokens.T.reshape(K, B)

    num_accepted, cur_tok0, npt, src_idx, cur_h0 = _sc_accept(
        cat_t, draft_t, slot_ids, cat_h)
    new_pool_tokens = npt.reshape(M, K)
    new_pool_hidden = _sc_pool_hidden(cat_h, src_idx).reshape(M, K, D)

    toks = _drafter_all(target.reshape(R), draft_tokens.reshape(B * K),
                        W_mtp, lm_head, embed_tokens, hidden_states)
    next_draft_tokens = toks[:, :, 0].T
    next_new_tokens = jnp.concatenate([cur_tok0[:, None], next_draft_tokens],
                                      axis=1)
    return (accepted_tokens, num_accepted, new_pool_hidden, new_pool_tokens,
            next_draft_tokens, next_new_tokens)
